# baseline (speedup 1.0000x reference)
import functools

import jax
import jax.numpy as jnp
from jax import lax
from jax.experimental import pallas as pl
from jax.experimental.pallas import tpu as pltpu
from jax.experimental.pallas import tpu_sc as plsc

_B = 4096
_C = 26
_D = 1024
_K = 32
_L = 16
_NC = 2
_NS = 16
_NW = _NC * _NS
_BPW = _B // _NW
_G = 32
_NG = _BPW // _G


def _rev(x):
    return lax.rev(x, (0,))


def _sort16(x):
    return plsc.sort_key_val(x, x)[0]


def _merge_pair(v0, v1, R0, R1):
    a = _sort16(v0)
    b = _sort16(v1)
    rb = _rev(b)
    u0 = _sort16(jnp.minimum(a, rb))
    u1 = _sort16(jnp.maximum(a, rb))
    m0 = jnp.minimum(R0, _rev(u1))
    m1 = jnp.minimum(R1, _rev(u0))
    lo = jnp.minimum(m0, m1)
    hi = jnp.maximum(m0, m1)
    return _sort16(lo), _sort16(hi)


def _sc_body(table, labels, out, idx_v, lab_v, buf_a, buf_b, out_v, sem_a, sem_b):
    wid = lax.axis_index("s") * _NC + lax.axis_index("c")
    base = wid * _BPW

    pltpu.sync_copy(labels.at[pl.ds(base, _BPW)], lab_v)
    for j in range(_BPW // _L):
        lab = lab_v[pl.ds(j * _L, _L)]
        rows = base + j * _L + lax.iota(jnp.int32, _L)
        idx_v[j // 2, pl.ds((j % 2) * _L, _L)] = rows * _C + lab

    def start(g, buf, sem):
        return pltpu.async_copy(table.at[idx_v.at[g]], buf, sem)

    def process(buf, g):
        inf16 = jnp.full((_L,), jnp.inf, jnp.float32)

        def row_body(r, carry):
            def chunk_body(c, R):
                Ra0, Ra1, Rb0, Rb1 = R
                off = c * 64
                Ra0, Ra1 = _merge_pair(
                    buf[r, pl.ds(off, _L)], buf[r, pl.ds(off + 16, _L)], Ra0, Ra1)
                Rb0, Rb1 = _merge_pair(
                    buf[r, pl.ds(off + 32, _L)], buf[r, pl.ds(off + 48, _L)], Rb0, Rb1)
                return (Ra0, Ra1, Rb0, Rb1)

            Ra0, Ra1, Rb0, Rb1 = lax.fori_loop(
                0, _D // 64, chunk_body, (inf16, inf16, inf16, inf16))
            m0 = jnp.minimum(Ra0, _rev(Rb1))
            m1 = jnp.minimum(Ra1, _rev(Rb0))
            lo = jnp.minimum(m0, m1)
            hi = jnp.maximum(m0, m1)
            out_v[g * _G + r, pl.ds(0, _L)] = _sort16(lo)
            out_v[g * _G + r, pl.ds(_L, _L)] = _sort16(hi)
            return carry

        lax.fori_loop(0, _G, row_body, jnp.int32(0))

    bufs = (buf_a, buf_b)
    sems = (sem_a, sem_b)
    cpy = {0: start(0, bufs[0], sems[0])}
    for g in range(_NG):
        if g + 1 < _NG:
            cpy[g + 1] = start(g + 1, bufs[(g + 1) % 2], sems[(g + 1) % 2])
        cpy[g].wait()
        process(bufs[g % 2], g)

    pltpu.sync_copy(out_v, out.at[pl.ds(base, _BPW)])


@functools.partial(
    pl.kernel,
    mesh=plsc.VectorSubcoreMesh(core_axis_name="c", subcore_axis_name="s"),
    out_type=jax.ShapeDtypeStruct((_B, _K), jnp.float32),
    compiler_params=pltpu.CompilerParams(needs_layout_passes=False),
    scratch_types=[
        pltpu.VMEM((_NG, _G), jnp.int32),
        pltpu.VMEM((_BPW,), jnp.int32),
        pltpu.VMEM((_G, _D), jnp.float32),
        pltpu.VMEM((_G, _D), jnp.float32),
        pltpu.VMEM((_BPW, _K), jnp.float32),
        pltpu.SemaphoreType.DMA,
        pltpu.SemaphoreType.DMA,
    ],
)
def _bottom_k_sc(table, labels, out, idx_v, lab_v, buf_a, buf_b, out_v, sem_a, sem_b):
    _sc_body(table, labels, out, idx_v, lab_v, buf_a, buf_b, out_v, sem_a, sem_b)


def _softmax_rows(x):
    m = jnp.max(x, axis=1, keepdims=True)
    e = jnp.exp(x - m)
    return e / jnp.sum(e, axis=1, keepdims=True)


def _kl_body(rho_ref, rhohat_ref, out_ref):
    p = _softmax_rows(rho_ref[...])
    q = _softmax_rows(rhohat_ref[...])
    s1 = jnp.sum(p * jnp.log(p / q))
    s2 = jnp.sum((1.0 - p) * jnp.log((1.0 - p) / (1.0 - q)))
    out_ref[0, 0] = s1 + s2


def kernel(rho, encoded, labels, K):
    table = encoded.reshape(_B * _C, _D)
    labels32 = labels.astype(jnp.int32)
    rho_hat = _bottom_k_sc(table, labels32)
    loss = pl.pallas_call(
        _kl_body,
        out_shape=jax.ShapeDtypeStruct((1, 1), jnp.float32),
        out_specs=pl.BlockSpec(memory_space=pltpu.SMEM),
    )(rho, rho_hat)
    return loss[0, 0]


# speedup vs baseline: 1.7443x; 1.7443x over previous
import functools

import jax
import jax.numpy as jnp
from jax import lax
from jax.experimental import pallas as pl
from jax.experimental.pallas import tpu as pltpu
from jax.experimental.pallas import tpu_sc as plsc

_B = 4096
_C = 26
_D = 1024
_K = 32
_L = 16
_NC = 2
_NS = 16
_NW = _NC * _NS
_BPW = _B // _NW
_G = 32
_NG = _BPW // _G


def _rev(x):
    return lax.rev(x, (0,))


def _sort16(x):
    return plsc.sort_key_val(x, x)[0]


def _merge_pair(v0, v1, R0, R1):
    a = _sort16(v0)
    b = _sort16(v1)
    rb = _rev(b)
    u0 = _sort16(jnp.minimum(a, rb))
    u1 = _sort16(jnp.maximum(a, rb))
    m0 = jnp.minimum(R0, _rev(u1))
    m1 = jnp.minimum(R1, _rev(u0))
    lo = jnp.minimum(m0, m1)
    hi = jnp.maximum(m0, m1)
    return _sort16(lo), _sort16(hi)


def _sc_body(encoded, labels, out, lab_v, buf_a, buf_b, out_v, sem_a, sem_b):
    wid = lax.axis_index("s") * _NC + lax.axis_index("c")
    base = wid * _BPW

    pltpu.sync_copy(labels.at[pl.ds(base, _BPW)], lab_v)
    lane = lax.iota(jnp.int32, _L)

    def start(g, buf, sem):
        cps = []
        for j in range(_G // _L):
            chunk = lab_v[pl.ds(g * _G + j * _L, _L)]
            for l in range(_L):
                lab = jnp.max(jnp.where(lane == l, chunk, jnp.int32(-1)))
                r = j * _L + l
                cps.append(pltpu.async_copy(
                    encoded.at[base + g * _G + r, lab], buf.at[r], sem))
        return cps

    def process(buf, g):
        inf16 = jnp.full((_L,), jnp.inf, jnp.float32)

        def row_body(r, carry):
            def chunk_body(c, R):
                Ra0, Ra1, Rb0, Rb1 = R
                off = c * 64
                Ra0, Ra1 = _merge_pair(
                    buf[r, pl.ds(off, _L)], buf[r, pl.ds(off + 16, _L)], Ra0, Ra1)
                Rb0, Rb1 = _merge_pair(
                    buf[r, pl.ds(off + 32, _L)], buf[r, pl.ds(off + 48, _L)], Rb0, Rb1)
                return (Ra0, Ra1, Rb0, Rb1)

            Ra0, Ra1, Rb0, Rb1 = lax.fori_loop(
                0, _D // 64, chunk_body, (inf16, inf16, inf16, inf16))
            m0 = jnp.minimum(Ra0, _rev(Rb1))
            m1 = jnp.minimum(Ra1, _rev(Rb0))
            lo = jnp.minimum(m0, m1)
            hi = jnp.maximum(m0, m1)
            out_v[g * _G + r, pl.ds(0, _L)] = _sort16(lo)
            out_v[g * _G + r, pl.ds(_L, _L)] = _sort16(hi)
            return carry

        lax.fori_loop(0, _G, row_body, jnp.int32(0))

    bufs = (buf_a, buf_b)
    sems = (sem_a, sem_b)
    cpy = {0: start(0, bufs[0], sems[0])}
    for g in range(_NG):
        if g + 1 < _NG:
            cpy[g + 1] = start(g + 1, bufs[(g + 1) % 2], sems[(g + 1) % 2])
        for c in cpy[g]:
            c.wait()
        process(bufs[g % 2], g)

    pltpu.sync_copy(out_v, out.at[pl.ds(base, _BPW)])


@functools.partial(
    pl.kernel,
    mesh=plsc.VectorSubcoreMesh(core_axis_name="c", subcore_axis_name="s"),
    out_type=jax.ShapeDtypeStruct((_B, _K), jnp.float32),
    compiler_params=pltpu.CompilerParams(needs_layout_passes=False),
    scratch_types=[
        pltpu.VMEM((_BPW,), jnp.int32),
        pltpu.VMEM((_G, _D), jnp.float32),
        pltpu.VMEM((_G, _D), jnp.float32),
        pltpu.VMEM((_BPW, _K), jnp.float32),
        pltpu.SemaphoreType.DMA,
        pltpu.SemaphoreType.DMA,
    ],
)
def _bottom_k_sc(encoded, labels, out, lab_v, buf_a, buf_b, out_v, sem_a, sem_b):
    _sc_body(encoded, labels, out, lab_v, buf_a, buf_b, out_v, sem_a, sem_b)


def _softmax_rows(x):
    m = jnp.max(x, axis=1, keepdims=True)
    e = jnp.exp(x - m)
    return e / jnp.sum(e, axis=1, keepdims=True)


def _kl_body(rho_ref, rhohat_ref, out_ref):
    p = _softmax_rows(rho_ref[...])
    q = _softmax_rows(rhohat_ref[...])
    s1 = jnp.sum(p * jnp.log(p / q))
    s2 = jnp.sum((1.0 - p) * jnp.log((1.0 - p) / (1.0 - q)))
    out_ref[0, 0] = s1 + s2


def kernel(rho, encoded, labels, K):
    labels32 = labels.astype(jnp.int32)
    rho_hat = _bottom_k_sc(encoded, labels32)
    loss = pl.pallas_call(
        _kl_body,
        out_shape=jax.ShapeDtypeStruct((1, 1), jnp.float32),
        out_specs=pl.BlockSpec(memory_space=pltpu.SMEM),
    )(rho, rho_hat)
    return loss[0, 0]


# speedup vs baseline: 1.7447x; 1.0003x over previous
import functools

import jax
import jax.numpy as jnp
from jax import lax
from jax.experimental import pallas as pl
from jax.experimental.pallas import tpu as pltpu
from jax.experimental.pallas import tpu_sc as plsc

_B = 4096
_C = 26
_D = 1024
_K = 32
_L = 16
_NC = 2
_NS = 16
_NW = _NC * _NS
_BPW = _B // _NW
_G = 32
_NG = _BPW // _G


def _rev(x):
    return lax.rev(x, (0,))


def _sort16(x):
    return plsc.sort_key_val(x, x)[0]


def _merge_pair(v0, v1, R0, R1):
    a = _sort16(v0)
    b = _sort16(v1)
    rb = _rev(b)
    u0 = _sort16(jnp.minimum(a, rb))
    u1 = _sort16(jnp.maximum(a, rb))
    m0 = jnp.minimum(R0, _rev(u1))
    m1 = jnp.minimum(R1, _rev(u0))
    lo = jnp.minimum(m0, m1)
    hi = jnp.maximum(m0, m1)
    return _sort16(lo), _sort16(hi)


def _sc_body(encoded, labels, out, lab_v, buf_a, buf_b, out_v, sem_a, sem_b):
    wid = lax.axis_index("s") * _NC + lax.axis_index("c")
    base = wid * _BPW

    pltpu.sync_copy(labels.at[pl.ds(base, _BPW)], lab_v)
    lane = lax.iota(jnp.int32, _L)

    def start(g, buf, sem):
        cps = []
        for j in range(_G // _L):
            chunk = lab_v[pl.ds(g * _G + j * _L, _L)]
            for l in range(_L):
                lab = jnp.max(jnp.where(lane == l, chunk, jnp.int32(-1)))
                r = j * _L + l
                cps.append(pltpu.async_copy(
                    encoded.at[base + g * _G + r, lab], buf.at[r], sem))
        return cps

    def process(buf, g):
        inf16 = jnp.full((_L,), jnp.inf, jnp.float32)

        def row_body(r, carry):
            def chunk_body(c, R):
                Ra0, Ra1, Rb0, Rb1 = R
                off = c * 64
                Ra0, Ra1 = _merge_pair(
                    buf[r, pl.ds(off, _L)], buf[r, pl.ds(off + 16, _L)], Ra0, Ra1)
                Rb0, Rb1 = _merge_pair(
                    buf[r, pl.ds(off + 32, _L)], buf[r, pl.ds(off + 48, _L)], Rb0, Rb1)
                return (Ra0, Ra1, Rb0, Rb1)

            Ra0, Ra1, Rb0, Rb1 = lax.fori_loop(
                0, _D // 64, chunk_body, (inf16, inf16, inf16, inf16))
            m0 = jnp.minimum(Ra0, _rev(Rb1))
            m1 = jnp.minimum(Ra1, _rev(Rb0))
            lo = jnp.minimum(m0, m1)
            hi = jnp.maximum(m0, m1)
            out_v[g * _G + r, pl.ds(0, _L)] = _sort16(lo)
            out_v[g * _G + r, pl.ds(_L, _L)] = _sort16(hi)
            return carry

        lax.fori_loop(0, _G, row_body, jnp.int32(0))

    bufs = (buf_a, buf_b)
    sems = (sem_a, sem_b)
    cpy = {0: start(0, bufs[0], sems[0])}
    for g in range(_NG):
        if g + 1 < _NG:
            cpy[g + 1] = start(g + 1, bufs[(g + 1) % 2], sems[(g + 1) % 2])
        for c in cpy[g]:
            c.wait()
        process(bufs[g % 2], g)

    pltpu.sync_copy(out_v, out.at[pl.ds(base, _BPW)])


@functools.partial(
    pl.kernel,
    mesh=plsc.VectorSubcoreMesh(core_axis_name="c", subcore_axis_name="s"),
    out_type=jax.ShapeDtypeStruct((_B, _K), jnp.float32),
    compiler_params=pltpu.CompilerParams(
        needs_layout_passes=False, use_tc_tiling_on_sc=True),
    scratch_types=[
        pltpu.VMEM((_BPW,), jnp.int32),
        pltpu.VMEM((_G, _D), jnp.float32),
        pltpu.VMEM((_G, _D), jnp.float32),
        pltpu.VMEM((_BPW, _K), jnp.float32),
        pltpu.SemaphoreType.DMA,
        pltpu.SemaphoreType.DMA,
    ],
)
def _bottom_k_sc(encoded, labels, out, lab_v, buf_a, buf_b, out_v, sem_a, sem_b):
    _sc_body(encoded, labels, out, lab_v, buf_a, buf_b, out_v, sem_a, sem_b)


def _softmax_rows(x):
    m = jnp.max(x, axis=1, keepdims=True)
    e = jnp.exp(x - m)
    return e / jnp.sum(e, axis=1, keepdims=True)


def _kl_body(rho_ref, rhohat_ref, out_ref):
    p = _softmax_rows(rho_ref[...])
    q = _softmax_rows(rhohat_ref[...])
    s1 = jnp.sum(p * jnp.log(p / q))
    s2 = jnp.sum((1.0 - p) * jnp.log((1.0 - p) / (1.0 - q)))
    out_ref[0, 0] = s1 + s2


def kernel(rho, encoded, labels, K):
    labels32 = labels.astype(jnp.int32)
    rho_hat = _bottom_k_sc(encoded, labels32)
    loss = pl.pallas_call(
        _kl_body,
        out_shape=jax.ShapeDtypeStruct((1, 1), jnp.float32),
        out_specs=pl.BlockSpec(memory_space=pltpu.SMEM),
    )(rho, rho_hat)
    return loss[0, 0]


# speedup vs baseline: 11.2037x; 6.4215x over previous
import functools

import jax
import jax.numpy as jnp
from jax import lax
from jax.experimental import pallas as pl
from jax.experimental.pallas import tpu as pltpu
from jax.experimental.pallas import tpu_sc as plsc

_B = 4096
_C = 26
_D = 1024
_K = 32
_L = 16
_NC = 2
_NS = 16
_NW = _NC * _NS
_BPW = _B // _NW
_G = 32
_NG = _BPW // _G


def _rev(x):
    return lax.rev(x, (0,))


def _sort16(x):
    return plsc.sort_key_val(x, x)[0]


def _merge_pair(v0, v1, R0, R1):
    a = _sort16(v0)
    b = _sort16(v1)
    rb = _rev(b)
    u0 = _sort16(jnp.minimum(a, rb))
    u1 = _sort16(jnp.maximum(a, rb))
    m0 = jnp.minimum(R0, _rev(u1))
    m1 = jnp.minimum(R1, _rev(u0))
    lo = jnp.minimum(m0, m1)
    hi = jnp.maximum(m0, m1)
    return _sort16(lo), _sort16(hi)


def _sc_body(encoded, labels, out, lab_v, buf_a, buf_b, out_v, sem_a, sem_b):
    wid = lax.axis_index("s") * _NC + lax.axis_index("c")
    base = wid * _BPW

    pltpu.sync_copy(labels.at[pl.ds(base, _BPW)], lab_v)
    lane = lax.iota(jnp.int32, _L)

    def start(g, buf, sem):
        cps = []
        for j in range(_G // _L):
            chunk = lab_v[pl.ds(g * _G + j * _L, _L)]
            for l in range(_L):
                lab = jnp.max(jnp.where(lane == l, chunk, jnp.int32(-1)))
                r = j * _L + l
                cps.append(pltpu.async_copy(
                    encoded.at[lab, base + g * _G + r], buf.at[r], sem))
        return cps

    def process(buf, g):
        inf16 = jnp.full((_L,), jnp.inf, jnp.float32)

        def row_body(r, carry):
            def chunk_body(c, R):
                Ra0, Ra1, Rb0, Rb1 = R
                off = c * 64
                Ra0, Ra1 = _merge_pair(
                    buf[r, pl.ds(off, _L)], buf[r, pl.ds(off + 16, _L)], Ra0, Ra1)
                Rb0, Rb1 = _merge_pair(
                    buf[r, pl.ds(off + 32, _L)], buf[r, pl.ds(off + 48, _L)], Rb0, Rb1)
                return (Ra0, Ra1, Rb0, Rb1)

            Ra0, Ra1, Rb0, Rb1 = lax.fori_loop(
                0, _D // 64, chunk_body, (inf16, inf16, inf16, inf16))
            m0 = jnp.minimum(Ra0, _rev(Rb1))
            m1 = jnp.minimum(Ra1, _rev(Rb0))
            lo = jnp.minimum(m0, m1)
            hi = jnp.maximum(m0, m1)
            out_v[g * _G + r, pl.ds(0, _L)] = _sort16(lo)
            out_v[g * _G + r, pl.ds(_L, _L)] = _sort16(hi)
            return carry

        lax.fori_loop(0, _G, row_body, jnp.int32(0))

    bufs = (buf_a, buf_b)
    sems = (sem_a, sem_b)
    cpy = {0: start(0, bufs[0], sems[0])}
    for g in range(_NG):
        if g + 1 < _NG:
            cpy[g + 1] = start(g + 1, bufs[(g + 1) % 2], sems[(g + 1) % 2])
        for c in cpy[g]:
            c.wait()
        process(bufs[g % 2], g)

    pltpu.sync_copy(out_v, out.at[pl.ds(base, _BPW)])


@functools.partial(
    pl.kernel,
    mesh=plsc.VectorSubcoreMesh(core_axis_name="c", subcore_axis_name="s"),
    out_type=jax.ShapeDtypeStruct((_B, _K), jnp.float32),
    compiler_params=pltpu.CompilerParams(needs_layout_passes=False),
    scratch_types=[
        pltpu.VMEM((_BPW,), jnp.int32),
        pltpu.VMEM((_G, _D), jnp.float32),
        pltpu.VMEM((_G, _D), jnp.float32),
        pltpu.VMEM((_BPW, _K), jnp.float32),
        pltpu.SemaphoreType.DMA,
        pltpu.SemaphoreType.DMA,
    ],
)
def _bottom_k_sc(encoded, labels, out, lab_v, buf_a, buf_b, out_v, sem_a, sem_b):
    _sc_body(encoded, labels, out, lab_v, buf_a, buf_b, out_v, sem_a, sem_b)


def _softmax_rows(x):
    m = jnp.max(x, axis=1, keepdims=True)
    e = jnp.exp(x - m)
    return e / jnp.sum(e, axis=1, keepdims=True)


def _kl_body(rho_ref, rhohat_ref, out_ref):
    p = _softmax_rows(rho_ref[...])
    q = _softmax_rows(rhohat_ref[...])
    s1 = jnp.sum(p * jnp.log(p / q))
    s2 = jnp.sum((1.0 - p) * jnp.log((1.0 - p) / (1.0 - q)))
    out_ref[0, 0] = s1 + s2


def kernel(rho, encoded, labels, K):
    enc_t = jnp.swapaxes(encoded, 0, 1)
    labels32 = labels.astype(jnp.int32)
    rho_hat = _bottom_k_sc(enc_t, labels32)
    loss = pl.pallas_call(
        _kl_body,
        out_shape=jax.ShapeDtypeStruct((1, 1), jnp.float32),
        out_specs=pl.BlockSpec(memory_space=pltpu.SMEM),
    )(rho, rho_hat)
    return loss[0, 0]


# speedup vs baseline: 11.7064x; 1.0449x over previous
import functools

import jax
import jax.numpy as jnp
from jax import lax
from jax.experimental import pallas as pl
from jax.experimental.pallas import tpu as pltpu
from jax.experimental.pallas import tpu_sc as plsc

_B = 4096
_C = 26
_D = 1024
_K = 32
_L = 16
_NC = 2
_NS = 16
_NW = _NC * _NS
_BPW = _B // _NW
_G = 32
_NG = _BPW // _G


def _rev(x):
    return lax.rev(x, (0,))


def _sort16(x):
    return plsc.sort_key_val(x, x)[0]


def _merge_pair(v0, v1, R0, R1):
    a = _sort16(v0)
    b = _sort16(v1)
    rb = _rev(b)
    u0 = _sort16(jnp.minimum(a, rb))
    u1 = _sort16(jnp.maximum(a, rb))
    m0 = jnp.minimum(R0, _rev(u1))
    m1 = jnp.minimum(R1, _rev(u0))
    lo = jnp.minimum(m0, m1)
    hi = jnp.maximum(m0, m1)
    return _sort16(lo), _sort16(hi)


def _sc_body(encoded, labels, out, lab_v, buf_a, buf_b, out_v, sem_a, sem_b):
    wid = lax.axis_index("s") * _NC + lax.axis_index("c")
    base = wid * _BPW

    pltpu.sync_copy(labels.at[pl.ds(base, _BPW)], lab_v)
    lane = lax.iota(jnp.int32, _L)
    lane_hi = lane + jnp.int32(_L)

    def start(g, buf, sem):
        cps = []
        for j in range(_G // _L):
            chunk = lab_v[pl.ds(g * _G + j * _L, _L)]
            for l in range(_L):
                lab = jnp.max(jnp.where(lane == l, chunk, jnp.int32(-1)))
                r = j * _L + l
                cps.append(pltpu.async_copy(
                    encoded.at[lab, base + g * _G + r], buf.at[r], sem))
        return cps

    def process(buf, g):
        inf16 = jnp.full((_L,), jnp.inf, jnp.float32)

        def row_body(r, carry):
            def chunk_body(c, R):
                Ra0, Ra1, Rb0, Rb1 = R
                off = c * 64
                Ra0, Ra1 = _merge_pair(
                    buf[r, pl.ds(off, _L)], buf[r, pl.ds(off + 16, _L)], Ra0, Ra1)
                Rb0, Rb1 = _merge_pair(
                    buf[r, pl.ds(off + 32, _L)], buf[r, pl.ds(off + 48, _L)], Rb0, Rb1)
                return (Ra0, Ra1, Rb0, Rb1)

            Ra0, Ra1, Rb0, Rb1 = lax.fori_loop(
                0, _D // 64, chunk_body, (inf16, inf16, inf16, inf16))
            m0 = jnp.minimum(Ra0, _rev(Rb1))
            m1 = jnp.minimum(Ra1, _rev(Rb0))
            lo = jnp.minimum(m0, m1)
            hi = jnp.maximum(m0, m1)
            col = jnp.full((_L,), g * _G + r, jnp.int32)
            plsc.store_scatter(out_v, [lane, col], _sort16(lo))
            plsc.store_scatter(out_v, [lane_hi, col], _sort16(hi))
            return carry

        lax.fori_loop(0, _G, row_body, jnp.int32(0))

    bufs = (buf_a, buf_b)
    sems = (sem_a, sem_b)
    cpy = {0: start(0, bufs[0], sems[0])}
    for g in range(_NG):
        if g + 1 < _NG:
            cpy[g + 1] = start(g + 1, bufs[(g + 1) % 2], sems[(g + 1) % 2])
        for c in cpy[g]:
            c.wait()
        process(bufs[g % 2], g)

    pltpu.sync_copy(out_v, out.at[:, pl.ds(base, _BPW)])


@functools.partial(
    pl.kernel,
    mesh=plsc.VectorSubcoreMesh(core_axis_name="c", subcore_axis_name="s"),
    out_type=jax.ShapeDtypeStruct((_K, _B), jnp.float32),
    compiler_params=pltpu.CompilerParams(needs_layout_passes=False),
    scratch_types=[
        pltpu.VMEM((_BPW,), jnp.int32),
        pltpu.VMEM((_G, _D), jnp.float32),
        pltpu.VMEM((_G, _D), jnp.float32),
        pltpu.VMEM((_K, _BPW), jnp.float32),
        pltpu.SemaphoreType.DMA,
        pltpu.SemaphoreType.DMA,
    ],
)
def _bottom_k_sc(encoded, labels, out, lab_v, buf_a, buf_b, out_v, sem_a, sem_b):
    _sc_body(encoded, labels, out, lab_v, buf_a, buf_b, out_v, sem_a, sem_b)


def _softmax_cols(x):
    m = jnp.max(x, axis=0, keepdims=True)
    e = jnp.exp(x - m)
    return e / jnp.sum(e, axis=0, keepdims=True)


def _kl_body(rho_ref, rhohat_ref, out_ref):
    p = _softmax_cols(rho_ref[...])
    q = _softmax_cols(rhohat_ref[...])
    s1 = jnp.sum(p * jnp.log(p / q))
    s2 = jnp.sum((1.0 - p) * jnp.log((1.0 - p) / (1.0 - q)))
    out_ref[0, 0] = s1 + s2


def kernel(rho, encoded, labels, K):
    enc_t = jnp.swapaxes(encoded, 0, 1)
    labels32 = labels.astype(jnp.int32)
    rho_hat_t = _bottom_k_sc(enc_t, labels32)
    rho_t = jnp.swapaxes(rho, 0, 1)
    loss = pl.pallas_call(
        _kl_body,
        out_shape=jax.ShapeDtypeStruct((1, 1), jnp.float32),
        out_specs=pl.BlockSpec(memory_space=pltpu.SMEM),
    )(rho_t, rho_hat_t)
    return loss[0, 0]


# speedup vs baseline: 13.6910x; 1.1695x over previous
import functools

import jax
import jax.numpy as jnp
from jax import lax
from jax.experimental import pallas as pl
from jax.experimental.pallas import tpu as pltpu
from jax.experimental.pallas import tpu_sc as plsc

_B = 4096
_C = 26
_D = 1024
_K = 32
_L = 16
_NC = 2
_NS = 16
_NW = _NC * _NS
_BPW = _B // _NW
_G = 32
_NG = _BPW // _G


def _rev(x):
    return lax.rev(x, (0,))


def _sort16(x):
    return plsc.sort_key_val(x, x)[0]


def _merge_pair(v0, v1, R0, R1):
    a = _sort16(v0)
    b = _sort16(v1)
    rb = _rev(b)
    u0 = _sort16(jnp.minimum(a, rb))
    u1 = _sort16(jnp.maximum(a, rb))
    m0 = jnp.minimum(R0, _rev(u1))
    m1 = jnp.minimum(R1, _rev(u0))
    lo = jnp.minimum(m0, m1)
    hi = jnp.maximum(m0, m1)
    return _sort16(lo), _sort16(hi)


def _sc_body(encoded, labels, out, lab_v, buf_a, buf_b, out_v, sem_a, sem_b):
    wid = lax.axis_index("s") * _NC + lax.axis_index("c")
    base = wid * _BPW

    pltpu.sync_copy(labels.at[pl.ds(base, _BPW)], lab_v)
    lane = lax.iota(jnp.int32, _L)
    lane_hi = lane + jnp.int32(_L)

    def start(g, buf, sem):
        cps = []
        for j in range(_G // _L):
            chunk = lab_v[pl.ds(g * _G + j * _L, _L)]
            for l in range(_L):
                lab = jnp.max(jnp.where(lane == l, chunk, jnp.int32(-1)))
                r = j * _L + l
                cps.append(pltpu.async_copy(
                    encoded.at[lab, base + g * _G + r], buf.at[r], sem))
        return cps

    def process(buf, g):
        inf16 = jnp.full((_L,), jnp.inf, jnp.float32)

        def finish(R, r):
            Ra0, Ra1, Rb0, Rb1 = R
            m0 = jnp.minimum(Ra0, _rev(Rb1))
            m1 = jnp.minimum(Ra1, _rev(Rb0))
            lo = jnp.minimum(m0, m1)
            hi = jnp.maximum(m0, m1)
            col = jnp.full((_L,), g * _G + r, jnp.int32)
            plsc.store_scatter(out_v, [lane, col], _sort16(lo))
            plsc.store_scatter(out_v, [lane_hi, col], _sort16(hi))

        def row_body(r, carry):
            r0 = 2 * r
            r1 = 2 * r + 1

            def chunk_body(c, R):
                RA, RB = R
                off = c * 64

                def quad(row, Rq):
                    q0, q1 = _merge_pair(
                        buf[row, pl.ds(off, _L)],
                        buf[row, pl.ds(off + 16, _L)], Rq[0], Rq[1])
                    q2, q3 = _merge_pair(
                        buf[row, pl.ds(off + 32, _L)],
                        buf[row, pl.ds(off + 48, _L)], Rq[2], Rq[3])
                    return (q0, q1, q2, q3)

                return (quad(r0, RA), quad(r1, RB))

            init = ((inf16, inf16, inf16, inf16), (inf16, inf16, inf16, inf16))
            RA, RB = lax.fori_loop(0, _D // 64, chunk_body, init)
            finish(RA, r0)
            finish(RB, r1)
            return carry

        lax.fori_loop(0, _G // 2, row_body, jnp.int32(0))

    bufs = (buf_a, buf_b)
    sems = (sem_a, sem_b)
    cpy = {0: start(0, bufs[0], sems[0])}
    for g in range(_NG):
        if g + 1 < _NG:
            cpy[g + 1] = start(g + 1, bufs[(g + 1) % 2], sems[(g + 1) % 2])
        for c in cpy[g]:
            c.wait()
        process(bufs[g % 2], g)

    pltpu.sync_copy(out_v, out.at[:, pl.ds(base, _BPW)])


@functools.partial(
    pl.kernel,
    mesh=plsc.VectorSubcoreMesh(core_axis_name="c", subcore_axis_name="s"),
    out_type=jax.ShapeDtypeStruct((_K, _B), jnp.float32),
    compiler_params=pltpu.CompilerParams(needs_layout_passes=False),
    scratch_types=[
        pltpu.VMEM((_BPW,), jnp.int32),
        pltpu.VMEM((_G, _D), jnp.float32),
        pltpu.VMEM((_G, _D), jnp.float32),
        pltpu.VMEM((_K, _BPW), jnp.float32),
        pltpu.SemaphoreType.DMA,
        pltpu.SemaphoreType.DMA,
    ],
)
def _bottom_k_sc(encoded, labels, out, lab_v, buf_a, buf_b, out_v, sem_a, sem_b):
    _sc_body(encoded, labels, out, lab_v, buf_a, buf_b, out_v, sem_a, sem_b)


def _softmax_cols(x):
    m = jnp.max(x, axis=0, keepdims=True)
    e = jnp.exp(x - m)
    return e / jnp.sum(e, axis=0, keepdims=True)


def _kl_body(rho_ref, rhohat_ref, out_ref):
    p = _softmax_cols(rho_ref[...])
    q = _softmax_cols(rhohat_ref[...])
    s1 = jnp.sum(p * jnp.log(p / q))
    s2 = jnp.sum((1.0 - p) * jnp.log((1.0 - p) / (1.0 - q)))
    out_ref[0, 0] = s1 + s2


def kernel(rho, encoded, labels, K):
    enc_t = jnp.swapaxes(encoded, 0, 1)
    labels32 = labels.astype(jnp.int32)
    rho_hat_t = _bottom_k_sc(enc_t, labels32)
    rho_t = jnp.swapaxes(rho, 0, 1)
    loss = pl.pallas_call(
        _kl_body,
        out_shape=jax.ShapeDtypeStruct((1, 1), jnp.float32),
        out_specs=pl.BlockSpec(memory_space=pltpu.SMEM),
    )(rho_t, rho_hat_t)
    return loss[0, 0]


# speedup vs baseline: 13.9573x; 1.0195x over previous
import functools

import jax
import jax.numpy as jnp
from jax import lax
from jax.experimental import pallas as pl
from jax.experimental.pallas import tpu as pltpu
from jax.experimental.pallas import tpu_sc as plsc

_B = 4096
_C = 26
_D = 1024
_K = 32
_L = 16
_NC = 2
_NS = 16
_NW = _NC * _NS
_BPW = _B // _NW
_G = 32
_NG = _BPW // _G


def _rev(x):
    return lax.rev(x, (0,))


def _sort16(x):
    return plsc.sort_key_val(x, x)[0]


def _merge_pair(v0, v1, R0, R1):
    a = _sort16(v0)
    b = _sort16(v1)
    rb = _rev(b)
    u0 = _sort16(jnp.minimum(a, rb))
    u1 = _sort16(jnp.maximum(a, rb))
    m0 = jnp.minimum(R0, _rev(u1))
    m1 = jnp.minimum(R1, _rev(u0))
    lo = jnp.minimum(m0, m1)
    hi = jnp.maximum(m0, m1)
    return _sort16(lo), _sort16(hi)


def _sc_body(encoded, labels, out, lab_v, buf_a, buf_b, out_v, sem_a, sem_b):
    wid = lax.axis_index("s") * _NC + lax.axis_index("c")
    base = wid * _BPW

    pltpu.sync_copy(labels.at[pl.ds(base, _BPW)], lab_v)
    lane = lax.iota(jnp.int32, _L)
    lane_hi = lane + jnp.int32(_L)

    def start(g, buf, sem):
        cps = []
        for j in range(_G // _L):
            chunk = lab_v[pl.ds(g * _G + j * _L, _L)]
            for l in range(_L):
                lab = jnp.max(jnp.where(lane == l, chunk, jnp.int32(-1)))
                r = j * _L + l
                cps.append(pltpu.async_copy(
                    encoded.at[lab, base + g * _G + r], buf.at[r], sem))
        return cps

    def process(buf, g):
        inf16 = jnp.full((_L,), jnp.inf, jnp.float32)

        def finish(R, r):
            Ra0, Ra1, Rb0, Rb1 = R
            m0 = jnp.minimum(Ra0, _rev(Rb1))
            m1 = jnp.minimum(Ra1, _rev(Rb0))
            lo = jnp.minimum(m0, m1)
            hi = jnp.maximum(m0, m1)
            col = jnp.full((_L,), g * _G + r, jnp.int32)
            plsc.store_scatter(out_v, [lane, col], _sort16(lo))
            plsc.store_scatter(out_v, [lane_hi, col], _sort16(hi))

        def row_body(r, carry):
            rows = [4 * r + k for k in range(4)]

            def chunk_body(c, R):
                off = c * 64

                def quad(row, Rq):
                    q0, q1 = _merge_pair(
                        buf[row, pl.ds(off, _L)],
                        buf[row, pl.ds(off + 16, _L)], Rq[0], Rq[1])
                    q2, q3 = _merge_pair(
                        buf[row, pl.ds(off + 32, _L)],
                        buf[row, pl.ds(off + 48, _L)], Rq[2], Rq[3])
                    return (q0, q1, q2, q3)

                return tuple(quad(row, Rq) for row, Rq in zip(rows, R))

            init = tuple((inf16, inf16, inf16, inf16) for _ in rows)
            R = lax.fori_loop(0, _D // 64, chunk_body, init)
            for Rq, row in zip(R, rows):
                finish(Rq, row)
            return carry

        lax.fori_loop(0, _G // 4, row_body, jnp.int32(0))

    bufs = (buf_a, buf_b)
    sems = (sem_a, sem_b)
    cpy = {0: start(0, bufs[0], sems[0])}
    for g in range(_NG):
        if g + 1 < _NG:
            cpy[g + 1] = start(g + 1, bufs[(g + 1) % 2], sems[(g + 1) % 2])
        for c in cpy[g]:
            c.wait()
        process(bufs[g % 2], g)

    pltpu.sync_copy(out_v, out.at[:, pl.ds(base, _BPW)])


@functools.partial(
    pl.kernel,
    mesh=plsc.VectorSubcoreMesh(core_axis_name="c", subcore_axis_name="s"),
    out_type=jax.ShapeDtypeStruct((_K, _B), jnp.float32),
    compiler_params=pltpu.CompilerParams(needs_layout_passes=False),
    scratch_types=[
        pltpu.VMEM((_BPW,), jnp.int32),
        pltpu.VMEM((_G, _D), jnp.float32),
        pltpu.VMEM((_G, _D), jnp.float32),
        pltpu.VMEM((_K, _BPW), jnp.float32),
        pltpu.SemaphoreType.DMA,
        pltpu.SemaphoreType.DMA,
    ],
)
def _bottom_k_sc(encoded, labels, out, lab_v, buf_a, buf_b, out_v, sem_a, sem_b):
    _sc_body(encoded, labels, out, lab_v, buf_a, buf_b, out_v, sem_a, sem_b)


def _softmax_cols(x):
    m = jnp.max(x, axis=0, keepdims=True)
    e = jnp.exp(x - m)
    return e / jnp.sum(e, axis=0, keepdims=True)


def _kl_body(rho_ref, rhohat_ref, out_ref):
    p = _softmax_cols(rho_ref[...])
    q = _softmax_cols(rhohat_ref[...])
    s1 = jnp.sum(p * jnp.log(p / q))
    s2 = jnp.sum((1.0 - p) * jnp.log((1.0 - p) / (1.0 - q)))
    out_ref[0, 0] = s1 + s2


def kernel(rho, encoded, labels, K):
    enc_t = jnp.swapaxes(encoded, 0, 1)
    labels32 = labels.astype(jnp.int32)
    rho_hat_t = _bottom_k_sc(enc_t, labels32)
    rho_t = jnp.swapaxes(rho, 0, 1)
    loss = pl.pallas_call(
        _kl_body,
        out_shape=jax.ShapeDtypeStruct((1, 1), jnp.float32),
        out_specs=pl.BlockSpec(memory_space=pltpu.SMEM),
    )(rho_t, rho_hat_t)
    return loss[0, 0]


# speedup vs baseline: 15.8118x; 1.1329x over previous
import functools

import jax
import jax.numpy as jnp
from jax import lax
from jax.experimental import pallas as pl
from jax.experimental.pallas import tpu as pltpu
from jax.experimental.pallas import tpu_sc as plsc

_B = 4096
_C = 26
_D = 1024
_K = 32
_L = 16
_NC = 2
_NS = 16
_NW = _NC * _NS
_BPW = _B // _NW
_G = 32
_NG = _BPW // _G


def _rev(x):
    return lax.rev(x, (0,))


def _sort16(x):
    return plsc.sort_key_val(x, x)[0]


def _merge_pair(v0, v1, R0, R1):
    a = _sort16(v0)
    b = _sort16(v1)
    rb = _rev(b)
    u0 = _sort16(jnp.minimum(a, rb))
    u1 = _sort16(jnp.maximum(a, rb))
    m0 = jnp.minimum(R0, _rev(u1))
    m1 = jnp.minimum(R1, _rev(u0))
    lo = jnp.minimum(m0, m1)
    hi = jnp.maximum(m0, m1)
    return _sort16(lo), _sort16(hi)


def _sc_body(encoded, labels, out, lab_v, buf_a, buf_b, out_v, sem_a, sem_b):
    wid = lax.axis_index("s") * _NC + lax.axis_index("c")
    base = wid * _BPW

    pltpu.sync_copy(labels.at[pl.ds(base, _BPW)], lab_v)
    lane = lax.iota(jnp.int32, _L)
    lane_hi = lane + jnp.int32(_L)

    def start(g, buf, sem):
        def issue(i, carry):
            chunk = lab_v[pl.ds(g * _G + (i // _L) * _L, _L)]
            lab = jnp.max(jnp.where(lane == i % _L, chunk, jnp.int32(-1)))
            pltpu.async_copy(encoded.at[lab, base + g * _G + i], buf.at[i], sem)
            return carry

        lax.fori_loop(0, _G, issue, jnp.int32(0))

    def drain(buf, sem):
        pltpu.make_async_copy(encoded.at[0, pl.ds(0, _G)], buf, sem).wait()

    def process(buf, g):
        inf16 = jnp.full((_L,), jnp.inf, jnp.float32)

        def finish(R, r):
            Ra0, Ra1, Rb0, Rb1 = R
            m0 = jnp.minimum(Ra0, _rev(Rb1))
            m1 = jnp.minimum(Ra1, _rev(Rb0))
            lo = jnp.minimum(m0, m1)
            hi = jnp.maximum(m0, m1)
            col = jnp.full((_L,), g * _G + r, jnp.int32)
            plsc.store_scatter(out_v, [lane, col], _sort16(lo))
            plsc.store_scatter(out_v, [lane_hi, col], _sort16(hi))

        def row_body(r, carry):
            rows = [4 * r + k for k in range(4)]

            def chunk_body(c, R):
                off = c * 64

                def quad(row, Rq):
                    q0, q1 = _merge_pair(
                        buf[row, pl.ds(off, _L)],
                        buf[row, pl.ds(off + 16, _L)], Rq[0], Rq[1])
                    q2, q3 = _merge_pair(
                        buf[row, pl.ds(off + 32, _L)],
                        buf[row, pl.ds(off + 48, _L)], Rq[2], Rq[3])
                    return (q0, q1, q2, q3)

                return tuple(quad(row, Rq) for row, Rq in zip(rows, R))

            init = tuple((inf16, inf16, inf16, inf16) for _ in rows)
            R = lax.fori_loop(0, _D // 64, chunk_body, init)
            for Rq, row in zip(R, rows):
                finish(Rq, row)
            return carry

        lax.fori_loop(0, _G // 4, row_body, jnp.int32(0))

    bufs = (buf_a, buf_b)
    sems = (sem_a, sem_b)
    start(0, bufs[0], sems[0])
    for g in range(_NG):
        if g + 1 < _NG:
            start(g + 1, bufs[(g + 1) % 2], sems[(g + 1) % 2])
        drain(bufs[g % 2], sems[g % 2])
        process(bufs[g % 2], g)

    pltpu.sync_copy(out_v, out.at[:, pl.ds(base, _BPW)])


@functools.partial(
    pl.kernel,
    mesh=plsc.VectorSubcoreMesh(core_axis_name="c", subcore_axis_name="s"),
    out_type=jax.ShapeDtypeStruct((_K, _B), jnp.float32),
    compiler_params=pltpu.CompilerParams(needs_layout_passes=False),
    scratch_types=[
        pltpu.VMEM((_BPW,), jnp.int32),
        pltpu.VMEM((_G, _D), jnp.float32),
        pltpu.VMEM((_G, _D), jnp.float32),
        pltpu.VMEM((_K, _BPW), jnp.float32),
        pltpu.SemaphoreType.DMA,
        pltpu.SemaphoreType.DMA,
    ],
)
def _bottom_k_sc(encoded, labels, out, lab_v, buf_a, buf_b, out_v, sem_a, sem_b):
    _sc_body(encoded, labels, out, lab_v, buf_a, buf_b, out_v, sem_a, sem_b)


def _softmax_cols(x):
    m = jnp.max(x, axis=0, keepdims=True)
    e = jnp.exp(x - m)
    return e / jnp.sum(e, axis=0, keepdims=True)


def _kl_body(rho_ref, rhohat_ref, out_ref):
    p = _softmax_cols(rho_ref[...])
    q = _softmax_cols(rhohat_ref[...])
    s1 = jnp.sum(p * jnp.log(p / q))
    s2 = jnp.sum((1.0 - p) * jnp.log((1.0 - p) / (1.0 - q)))
    out_ref[0, 0] = s1 + s2


def kernel(rho, encoded, labels, K):
    enc_t = jnp.swapaxes(encoded, 0, 1)
    labels32 = labels.astype(jnp.int32)
    rho_hat_t = _bottom_k_sc(enc_t, labels32)
    rho_t = jnp.swapaxes(rho, 0, 1)
    loss = pl.pallas_call(
        _kl_body,
        out_shape=jax.ShapeDtypeStruct((1, 1), jnp.float32),
        out_specs=pl.BlockSpec(memory_space=pltpu.SMEM),
    )(rho_t, rho_hat_t)
    return loss[0, 0]


# speedup vs baseline: 16.0539x; 1.0153x over previous
import functools

import jax
import jax.numpy as jnp
from jax import lax
from jax.experimental import pallas as pl
from jax.experimental.pallas import tpu as pltpu
from jax.experimental.pallas import tpu_sc as plsc

_B = 4096
_C = 26
_D = 1024
_K = 32
_L = 16
_NC = 2
_NS = 16
_NW = _NC * _NS
_BPW = _B // _NW
_G = 32
_NG = _BPW // _G


def _rev(x):
    return lax.rev(x, (0,))


def _sort16(x):
    return plsc.sort_key_val(x, x)[0]


def _merge_pair(v0, v1, R0, R1):
    a = _sort16(v0)
    b = _sort16(v1)
    rb = _rev(b)
    u0 = _sort16(jnp.minimum(a, rb))
    u1 = _sort16(jnp.maximum(a, rb))
    m0 = jnp.minimum(R0, _rev(u1))
    m1 = jnp.minimum(R1, _rev(u0))
    lo = jnp.minimum(m0, m1)
    hi = jnp.maximum(m0, m1)
    return _sort16(lo), _sort16(hi)


def _sc_body(encoded, labels, out, lab_v, buf_a, buf_b, out_v, sem_a, sem_b):
    wid = lax.axis_index("s") * _NC + lax.axis_index("c")
    base = wid * _BPW

    pltpu.sync_copy(labels.at[pl.ds(base, _BPW)], lab_v)
    lane = lax.iota(jnp.int32, _L)
    lane_hi = lane + jnp.int32(_L)

    def start(g, buf, sem):
        def issue(i, carry):
            chunk = lab_v[pl.ds(g * _G + (i // _L) * _L, _L)]
            lab = jnp.max(jnp.where(lane == i % _L, chunk, jnp.int32(-1)))
            pltpu.async_copy(encoded.at[lab, base + g * _G + i], buf.at[i], sem)
            return carry

        lax.fori_loop(0, _G, issue, jnp.int32(0))

    def drain(buf, sem):
        pltpu.make_async_copy(encoded.at[0, pl.ds(0, _G)], buf, sem).wait()

    def process(buf, g):
        inf16 = jnp.full((_L,), jnp.inf, jnp.float32)

        def finish(R, r):
            Ra0, Ra1, Rb0, Rb1 = R
            m0 = jnp.minimum(Ra0, _rev(Rb1))
            m1 = jnp.minimum(Ra1, _rev(Rb0))
            lo = jnp.minimum(m0, m1)
            hi = jnp.maximum(m0, m1)
            col = jnp.full((_L,), g * _G + r, jnp.int32)
            plsc.store_scatter(out_v, [lane, col], _sort16(lo))
            plsc.store_scatter(out_v, [lane_hi, col], _sort16(hi))

        def row_body(r, carry):
            rows = [4 * r + k for k in range(4)]

            def chunk_body(c, R):
                off = c * 64

                def quad(row, Rq):
                    q0, q1 = _merge_pair(
                        buf[row, pl.ds(off, _L)],
                        buf[row, pl.ds(off + 16, _L)], Rq[0], Rq[1])
                    q2, q3 = _merge_pair(
                        buf[row, pl.ds(off + 32, _L)],
                        buf[row, pl.ds(off + 48, _L)], Rq[2], Rq[3])
                    return (q0, q1, q2, q3)

                return tuple(quad(row, Rq) for row, Rq in zip(rows, R))

            init = tuple((inf16, inf16, inf16, inf16) for _ in rows)
            R = lax.fori_loop(0, _D // 64, chunk_body, init)
            for Rq, row in zip(R, rows):
                finish(Rq, row)
            return carry

        lax.fori_loop(0, _G // 4, row_body, jnp.int32(0))

    start(0, buf_a, sem_a)

    def group_pair(t, carry):
        g0 = 2 * t
        start(g0 + 1, buf_b, sem_b)
        drain(buf_a, sem_a)
        process(buf_a, g0)

        @pl.when(t + 1 < _NG // 2)
        def _():
            start(g0 + 2, buf_a, sem_a)

        drain(buf_b, sem_b)
        process(buf_b, g0 + 1)
        return carry

    lax.fori_loop(0, _NG // 2, group_pair, jnp.int32(0))

    pltpu.sync_copy(out_v, out.at[:, pl.ds(base, _BPW)])


@functools.partial(
    pl.kernel,
    mesh=plsc.VectorSubcoreMesh(core_axis_name="c", subcore_axis_name="s"),
    out_type=jax.ShapeDtypeStruct((_K, _B), jnp.float32),
    compiler_params=pltpu.CompilerParams(needs_layout_passes=False),
    scratch_types=[
        pltpu.VMEM((_BPW,), jnp.int32),
        pltpu.VMEM((_G, _D), jnp.float32),
        pltpu.VMEM((_G, _D), jnp.float32),
        pltpu.VMEM((_K, _BPW), jnp.float32),
        pltpu.SemaphoreType.DMA,
        pltpu.SemaphoreType.DMA,
    ],
)
def _bottom_k_sc(encoded, labels, out, lab_v, buf_a, buf_b, out_v, sem_a, sem_b):
    _sc_body(encoded, labels, out, lab_v, buf_a, buf_b, out_v, sem_a, sem_b)


def _softmax_cols(x):
    m = jnp.max(x, axis=0, keepdims=True)
    e = jnp.exp(x - m)
    return e / jnp.sum(e, axis=0, keepdims=True)


def _kl_body(rho_ref, rhohat_ref, out_ref):
    p = _softmax_cols(rho_ref[...])
    q = _softmax_cols(rhohat_ref[...])
    s1 = jnp.sum(p * jnp.log(p / q))
    s2 = jnp.sum((1.0 - p) * jnp.log((1.0 - p) / (1.0 - q)))
    out_ref[0, 0] = s1 + s2


def kernel(rho, encoded, labels, K):
    enc_t = jnp.swapaxes(encoded, 0, 1)
    labels32 = labels.astype(jnp.int32)
    rho_hat_t = _bottom_k_sc(enc_t, labels32)
    rho_t = jnp.swapaxes(rho, 0, 1)
    loss = pl.pallas_call(
        _kl_body,
        out_shape=jax.ShapeDtypeStruct((1, 1), jnp.float32),
        out_specs=pl.BlockSpec(memory_space=pltpu.SMEM),
    )(rho_t, rho_hat_t)
    return loss[0, 0]


# speedup vs baseline: 16.1094x; 1.0035x over previous
import functools

import jax
import jax.numpy as jnp
from jax import lax
from jax.experimental import pallas as pl
from jax.experimental.pallas import tpu as pltpu
from jax.experimental.pallas import tpu_sc as plsc

_B = 4096
_C = 26
_D = 1024
_K = 32
_L = 16
_NC = 2
_NS = 16
_NW = _NC * _NS
_BPW = _B // _NW
_G = 32
_NG = _BPW // _G


def _rev(x):
    return lax.rev(x, (0,))


def _sort16(x):
    return plsc.sort_key_val(x, x)[0]


def _merge_pair(v0, v1, R0, R1):
    a = _sort16(v0)
    b = _sort16(v1)
    rb = _rev(b)
    u0 = _sort16(jnp.minimum(a, rb))
    u1 = _sort16(jnp.maximum(a, rb))
    m0 = jnp.minimum(R0, _rev(u1))
    m1 = jnp.minimum(R1, _rev(u0))
    lo = jnp.minimum(m0, m1)
    hi = jnp.maximum(m0, m1)
    return _sort16(lo), _sort16(hi)


def _sc_body(encoded, labels, out, lab_v, bufs, out_v, sem_a, sem_b, sem_c):
    wid = lax.axis_index("s") * _NC + lax.axis_index("c")
    base = wid * _BPW

    pltpu.sync_copy(labels.at[pl.ds(base, _BPW)], lab_v)
    lane = lax.iota(jnp.int32, _L)
    lane_hi = lane + jnp.int32(_L)

    def start(g, m, sem):
        def issue(i, carry):
            chunk = lab_v[pl.ds(g * _G + (i // _L) * _L, _L)]
            lab = jnp.max(jnp.where(lane == i % _L, chunk, jnp.int32(-1)))
            pltpu.async_copy(encoded.at[lab, base + g * _G + i], bufs.at[m, i], sem)
            return carry

        lax.fori_loop(0, _G, issue, jnp.int32(0))

    def drain(m, sem):
        pltpu.make_async_copy(encoded.at[0, pl.ds(0, _G)], bufs.at[m], sem).wait()

    def process(m, g):
        inf16 = jnp.full((_L,), jnp.inf, jnp.float32)

        def finish(R, r):
            Ra0, Ra1, Rb0, Rb1 = R
            m0 = jnp.minimum(Ra0, _rev(Rb1))
            m1 = jnp.minimum(Ra1, _rev(Rb0))
            lo = jnp.minimum(m0, m1)
            hi = jnp.maximum(m0, m1)
            col = jnp.full((_L,), g * _G + r, jnp.int32)
            plsc.store_scatter(out_v, [lane, col], _sort16(lo))
            plsc.store_scatter(out_v, [lane_hi, col], _sort16(hi))

        def row_body(r, carry):
            rows = [4 * r + k for k in range(4)]

            def chunk_body(c, R):
                off = c * 64

                def quad(row, Rq):
                    q0, q1 = _merge_pair(
                        bufs[m, row, pl.ds(off, _L)],
                        bufs[m, row, pl.ds(off + 16, _L)], Rq[0], Rq[1])
                    q2, q3 = _merge_pair(
                        bufs[m, row, pl.ds(off + 32, _L)],
                        bufs[m, row, pl.ds(off + 48, _L)], Rq[2], Rq[3])
                    return (q0, q1, q2, q3)

                return tuple(quad(row, Rq) for row, Rq in zip(rows, R))

            init = tuple((inf16, inf16, inf16, inf16) for _ in rows)
            R = lax.fori_loop(0, _D // 64, chunk_body, init)
            for Rq, row in zip(R, rows):
                finish(Rq, row)
            return carry

        lax.fori_loop(0, _G // 4, row_body, jnp.int32(0))

    sems = (sem_a, sem_b, sem_c)

    def sem_switch(m, fn):
        for s in range(3):
            @pl.when(m == s)
            def _(s=s):
                fn(sems[s])

    start(0, jnp.int32(0), sem_a)
    start(1, jnp.int32(1), sem_b)

    def group(g, carry):
        m = lax.rem(g, 3)

        @pl.when(g + 2 < _NG)
        def _():
            m2 = lax.rem(g + 2, 3)
            sem_switch(m2, lambda s: start(g + 2, m2, s))

        sem_switch(m, lambda s: drain(m, s))
        process(m, g)
        return carry

    lax.fori_loop(0, _NG, group, jnp.int32(0))

    pltpu.sync_copy(out_v, out.at[:, pl.ds(base, _BPW)])


@functools.partial(
    pl.kernel,
    mesh=plsc.VectorSubcoreMesh(core_axis_name="c", subcore_axis_name="s"),
    out_type=jax.ShapeDtypeStruct((_K, _B), jnp.float32),
    compiler_params=pltpu.CompilerParams(needs_layout_passes=False),
    scratch_types=[
        pltpu.VMEM((_BPW,), jnp.int32),
        pltpu.VMEM((3, _G, _D), jnp.float32),
        pltpu.VMEM((_K, _BPW), jnp.float32),
        pltpu.SemaphoreType.DMA,
        pltpu.SemaphoreType.DMA,
        pltpu.SemaphoreType.DMA,
    ],
)
def _bottom_k_sc(encoded, labels, out, lab_v, bufs, out_v, sem_a, sem_b, sem_c):
    _sc_body(encoded, labels, out, lab_v, bufs, out_v, sem_a, sem_b, sem_c)


def _softmax_cols(x):
    m = jnp.max(x, axis=0, keepdims=True)
    e = jnp.exp(x - m)
    return e / jnp.sum(e, axis=0, keepdims=True)


def _kl_body(rho_ref, rhohat_ref, out_ref):
    p = _softmax_cols(rho_ref[...])
    q = _softmax_cols(rhohat_ref[...])
    s1 = jnp.sum(p * jnp.log(p / q))
    s2 = jnp.sum((1.0 - p) * jnp.log((1.0 - p) / (1.0 - q)))
    out_ref[0, 0] = s1 + s2


def kernel(rho, encoded, labels, K):
    enc_t = jnp.swapaxes(encoded, 0, 1)
    labels32 = labels.astype(jnp.int32)
    rho_hat_t = _bottom_k_sc(enc_t, labels32)
    rho_t = jnp.swapaxes(rho, 0, 1)
    loss = pl.pallas_call(
        _kl_body,
        out_shape=jax.ShapeDtypeStruct((1, 1), jnp.float32),
        out_specs=pl.BlockSpec(memory_space=pltpu.SMEM),
    )(rho_t, rho_hat_t)
    return loss[0, 0]
